# baseline (speedup 1.0000x reference)
import functools
import math

import jax
import jax.numpy as jnp
import numpy as np
from jax import lax
from jax.experimental import pallas as pl
from jax.experimental.pallas import tpu as pltpu
from jax.experimental.pallas import tpu_sc as plsc

BATCH = 4
MAX_SEQ = 8192
D_MODEL = 768
SCALE = 1.0 / math.sqrt(float(D_MODEL))

NC = 2
NS = 16
NW = NC * NS
S_PER_W = MAX_SEQ // NW
CHUNK = 8
N_CHUNKS = S_PER_W // CHUNK
NSETS = 4
N_QUADS = N_CHUNKS // NSETS
LANES = 16
VECS_PER_ROW = D_MODEL // LANES


def _make_pos_encoding():
    position = np.arange(MAX_SEQ, dtype=np.float32).reshape(MAX_SEQ, 1)
    even_index = np.arange(0, D_MODEL, 2).astype(np.float32)
    denominator = np.power(10000.0, even_index / float(D_MODEL))
    even_pos = np.sin(position / denominator)
    odd_pos = np.cos(position / denominator)
    pe = np.stack([even_pos, odd_pos], axis=2).reshape(MAX_SEQ, D_MODEL)
    return jnp.asarray(pe, dtype=jnp.float32)


_MESH = plsc.VectorSubcoreMesh(core_axis_name="c", subcore_axis_name="s")

_PACK_ROWS = 256


def _pack_body(x_ref, o_ref):
    a = x_ref[:, : D_MODEL // 2]
    b = x_ref[:, D_MODEL // 2:]
    ua = jax.lax.bitcast_convert_type(a, jnp.uint32)
    ub = jax.lax.bitcast_convert_type(b, jnp.uint32)
    half = jnp.uint32(0x7FFF)
    one = jnp.uint32(1)
    ra = (ua + half + ((ua >> jnp.uint32(16)) & one)) >> jnp.uint32(16)
    rb = (ub + half + ((ub >> jnp.uint32(16)) & one)) >> jnp.uint32(16)
    o_ref[...] = jax.lax.bitcast_convert_type(
        ra | (rb << jnp.uint32(16)), jnp.int32)


_pack_table = pl.pallas_call(
    _pack_body,
    out_shape=jax.ShapeDtypeStruct((MAX_SEQ, D_MODEL // 2), jnp.int32),
    grid=(MAX_SEQ // _PACK_ROWS,),
    in_specs=[pl.BlockSpec((_PACK_ROWS, D_MODEL), lambda i: (i, 0))],
    out_specs=pl.BlockSpec((_PACK_ROWS, D_MODEL // 2), lambda i: (i, 0)),
)


@functools.partial(
    pl.kernel,
    mesh=_MESH,
    out_type=jax.ShapeDtypeStruct((BATCH, MAX_SEQ, D_MODEL), jnp.float32),
    scratch_types=[
        pltpu.VMEM((BATCH, S_PER_W), jnp.int32),
        pltpu.VMEM((2, CHUNK, D_MODEL), jnp.float32),
        pltpu.VMEM((NSETS, BATCH, CHUNK, D_MODEL // 2), jnp.int32),
        pltpu.VMEM((2, BATCH, CHUNK, D_MODEL), jnp.float32),
    ] + [pltpu.SemaphoreType.DMA] * (NSETS + 2 + 2),
)
def _embed_kernel(x_hbm, table_hbm, pe_hbm, out_hbm,
                  idx_v, pe_v, g_v, o_v, *sems):
    gsem = sems[:NSETS]
    ssem = sems[NSETS:NSETS + 2]
    psem = sems[NSETS + 2:]
    wid = lax.axis_index("s") * NC + lax.axis_index("c")
    sbase = wid * S_PER_W

    for b in range(BATCH):
        pltpu.sync_copy(x_hbm.at[b, pl.ds(sbase, S_PER_W)], idx_v.at[b])

    def gather_start(c, s):
        for b in range(BATCH):
            pltpu.async_copy(
                table_hbm.at[idx_v.at[b, pl.ds(c * CHUNK, CHUNK)]],
                g_v.at[s, b], gsem[s])

    def gather_wait(c, s):
        for b in range(BATCH):
            pltpu.make_async_copy(
                table_hbm.at[pl.ds(0, CHUNK)], g_v.at[s, b],
                gsem[s]).wait()

    def pe_start(c, s):
        pltpu.async_copy(
            pe_hbm.at[pl.ds(sbase + c * CHUNK, CHUNK)], pe_v.at[s % 2],
            psem[s % 2])

    def pe_wait(c, s):
        pltpu.make_async_copy(
            pe_hbm.at[pl.ds(sbase + c * CHUNK, CHUNK)], pe_v.at[s % 2],
            psem[s % 2]).wait()

    def store_start(c, s):
        pltpu.async_copy(
            o_v.at[s], out_hbm.at[:, pl.ds(sbase + c * CHUNK, CHUNK)],
            ssem[s])

    def store_wait(c, s):
        pltpu.make_async_copy(
            o_v.at[s], out_hbm.at[:, pl.ds(sbase + c * CHUNK, CHUNK)],
            ssem[s]).wait()

    mask_hi = jnp.int32(-65536)

    def compute(gs, os, ps):
        def row_body(r, carry):
            for k in range(VECS_PER_ROW // 2):
                sl0 = pl.ds(k * LANES, LANES)
                sl1 = pl.ds((VECS_PER_ROW // 2 + k) * LANES, LANES)
                pe0 = pe_v[ps, r, sl0]
                pe1 = pe_v[ps, r, sl1]
                ws = [g_v[gs, b, r, pl.ds(k * LANES, LANES)]
                      for b in range(BATCH)]
                evens = [lax.bitcast_convert_type(w << 16, jnp.float32)
                         for w in ws]
                odds = [lax.bitcast_convert_type(w & mask_hi, jnp.float32)
                        for w in ws]
                res0 = [e * SCALE + pe0 for e in evens]
                res1 = [o * SCALE + pe1 for o in odds]
                for b in range(BATCH):
                    o_v[os, b, r, sl0] = res0[b]
                    o_v[os, b, r, sl1] = res1[b]
            return carry

        lax.fori_loop(0, CHUNK, row_body, 0)

    gather_start(0, 0)
    pe_start(0, 0)

    def quad_body(q, carry):
        for cc in range(NSETS):
            c = q * NSETS + cc
            gs, os, ps = cc, cc % 2, cc % 2
            ns = (cc + 1) % NSETS

            gather_wait(c, gs)
            pe_wait(c, ps)

            if cc == NSETS - 1:
                @pl.when(q < N_QUADS - 1)
                def _fire_next_last():
                    gather_start(c + 1, ns)
                    pe_start(c + 1, ns)
            else:
                gather_start(c + 1, ns)
                pe_start(c + 1, ns)

            if cc >= 2:
                store_wait(c - 2, os)
            else:
                @pl.when(q > 0)
                def _drain_old_store():
                    store_wait(c - 2, os)

            compute(gs, os, ps)
            store_start(c, os)
        return carry

    lax.fori_loop(0, N_QUADS, quad_body, 0)

    store_wait(N_CHUNKS - 2, 0)
    store_wait(N_CHUNKS - 1, 1)


def kernel(x, emb_table):
    packed = _pack_table(emb_table)
    pe = _make_pos_encoding()
    return _embed_kernel(x, packed, pe)


# speedup vs baseline: 1.0914x; 1.0914x over previous
import functools
import math

import jax
import jax.numpy as jnp
import numpy as np
from jax import lax
from jax.experimental import pallas as pl
from jax.experimental.pallas import tpu as pltpu
from jax.experimental.pallas import tpu_sc as plsc

BATCH = 4
MAX_SEQ = 8192
D_MODEL = 768
SCALE = 1.0 / math.sqrt(float(D_MODEL))

NC = 2
NS = 16
NW = NC * NS
S_PER_W = MAX_SEQ // NW
CHUNK = 8
N_CHUNKS = S_PER_W // CHUNK
NSETS = 4
N_QUADS = N_CHUNKS // NSETS
LANES = 16
VECS_PER_ROW = D_MODEL // LANES


def _make_pos_encoding():
    position = np.arange(MAX_SEQ, dtype=np.float32).reshape(MAX_SEQ, 1)
    even_index = np.arange(0, D_MODEL, 2).astype(np.float32)
    denominator = np.power(10000.0, even_index / float(D_MODEL))
    even_pos = np.sin(position / denominator)
    odd_pos = np.cos(position / denominator)
    pe = np.stack([even_pos, odd_pos], axis=2).reshape(MAX_SEQ, D_MODEL)
    u = pe.astype(np.float32).view(np.uint32)
    r = ((u + 0x7FFF + ((u >> 16) & 1)) >> 16).astype(np.uint32)
    lo = r[:, : D_MODEL // 2]
    hi = r[:, D_MODEL // 2:]
    packed = (lo | (hi << 16)).astype(np.uint32).view(np.int32)
    return jnp.asarray(packed)


_MESH = plsc.VectorSubcoreMesh(core_axis_name="c", subcore_axis_name="s")

_PACK_ROWS = 256


def _pack_body(x_ref, o_ref):
    a = x_ref[:, : D_MODEL // 2]
    b = x_ref[:, D_MODEL // 2:]
    ua = jax.lax.bitcast_convert_type(a, jnp.uint32)
    ub = jax.lax.bitcast_convert_type(b, jnp.uint32)
    half = jnp.uint32(0x7FFF)
    one = jnp.uint32(1)
    ra = (ua + half + ((ua >> jnp.uint32(16)) & one)) >> jnp.uint32(16)
    rb = (ub + half + ((ub >> jnp.uint32(16)) & one)) >> jnp.uint32(16)
    o_ref[...] = jax.lax.bitcast_convert_type(
        ra | (rb << jnp.uint32(16)), jnp.int32)


_pack_table = pl.pallas_call(
    _pack_body,
    out_shape=jax.ShapeDtypeStruct((MAX_SEQ, D_MODEL // 2), jnp.int32),
    grid=(MAX_SEQ // _PACK_ROWS,),
    in_specs=[pl.BlockSpec((_PACK_ROWS, D_MODEL), lambda i: (i, 0))],
    out_specs=pl.BlockSpec((_PACK_ROWS, D_MODEL // 2), lambda i: (i, 0)),
)


@functools.partial(
    pl.kernel,
    mesh=_MESH,
    out_type=jax.ShapeDtypeStruct((BATCH, MAX_SEQ, D_MODEL), jnp.float32),
    scratch_types=[
        pltpu.VMEM((BATCH, S_PER_W), jnp.int32),
        pltpu.VMEM((2, CHUNK, D_MODEL // 2), jnp.int32),
        pltpu.VMEM((NSETS, BATCH, CHUNK, D_MODEL // 2), jnp.int32),
        pltpu.VMEM((2, BATCH, CHUNK, D_MODEL), jnp.float32),
    ] + [pltpu.SemaphoreType.DMA] * (NSETS + 2 + 2),
)
def _embed_kernel(x_hbm, table_hbm, pe_hbm, out_hbm,
                  idx_v, pe_v, g_v, o_v, *sems):
    gsem = sems[:NSETS]
    ssem = sems[NSETS:NSETS + 2]
    psem = sems[NSETS + 2:]
    wid = lax.axis_index("s") * NC + lax.axis_index("c")
    sbase = wid * S_PER_W

    for b in range(BATCH):
        pltpu.sync_copy(x_hbm.at[b, pl.ds(sbase, S_PER_W)], idx_v.at[b])

    def gather_start(c, s):
        for b in range(BATCH):
            pltpu.async_copy(
                table_hbm.at[idx_v.at[b, pl.ds(c * CHUNK, CHUNK)]],
                g_v.at[s, b], gsem[s])

    def gather_wait(c, s):
        for b in range(BATCH):
            pltpu.make_async_copy(
                table_hbm.at[pl.ds(0, CHUNK)], g_v.at[s, b],
                gsem[s]).wait()

    def pe_start(c, s):
        pltpu.async_copy(
            pe_hbm.at[pl.ds(sbase + c * CHUNK, CHUNK)], pe_v.at[s % 2],
            psem[s % 2])

    def pe_wait(c, s):
        pltpu.make_async_copy(
            pe_hbm.at[pl.ds(sbase + c * CHUNK, CHUNK)], pe_v.at[s % 2],
            psem[s % 2]).wait()

    def store_start(c, s):
        pltpu.async_copy(
            o_v.at[s], out_hbm.at[:, pl.ds(sbase + c * CHUNK, CHUNK)],
            ssem[s])

    def store_wait(c, s):
        pltpu.make_async_copy(
            o_v.at[s], out_hbm.at[:, pl.ds(sbase + c * CHUNK, CHUNK)],
            ssem[s]).wait()

    mask_hi = jnp.int32(-65536)

    def compute(gs, os, ps):
        def row_body(r, carry):
            for k in range(VECS_PER_ROW // 2):
                sl0 = pl.ds(k * LANES, LANES)
                sl1 = pl.ds((VECS_PER_ROW // 2 + k) * LANES, LANES)
                wp = pe_v[ps, r, pl.ds(k * LANES, LANES)]
                pe0 = lax.bitcast_convert_type(wp << 16, jnp.float32)
                pe1 = lax.bitcast_convert_type(wp & mask_hi, jnp.float32)
                ws = [g_v[gs, b, r, pl.ds(k * LANES, LANES)]
                      for b in range(BATCH)]
                evens = [lax.bitcast_convert_type(w << 16, jnp.float32)
                         for w in ws]
                odds = [lax.bitcast_convert_type(w & mask_hi, jnp.float32)
                        for w in ws]
                res0 = [e * SCALE + pe0 for e in evens]
                res1 = [o * SCALE + pe1 for o in odds]
                for b in range(BATCH):
                    o_v[os, b, r, sl0] = res0[b]
                    o_v[os, b, r, sl1] = res1[b]
            return carry

        lax.fori_loop(0, CHUNK, row_body, 0)

    gather_start(0, 0)
    pe_start(0, 0)

    def quad_body(q, carry):
        for cc in range(NSETS):
            c = q * NSETS + cc
            gs, os, ps = cc, cc % 2, cc % 2
            ns = (cc + 1) % NSETS

            gather_wait(c, gs)
            pe_wait(c, ps)

            if cc == NSETS - 1:
                @pl.when(q < N_QUADS - 1)
                def _fire_next_last():
                    gather_start(c + 1, ns)
                    pe_start(c + 1, ns)
            else:
                gather_start(c + 1, ns)
                pe_start(c + 1, ns)

            if cc >= 2:
                store_wait(c - 2, os)
            else:
                @pl.when(q > 0)
                def _drain_old_store():
                    store_wait(c - 2, os)

            compute(gs, os, ps)
            store_start(c, os)
        return carry

    lax.fori_loop(0, N_QUADS, quad_body, 0)

    store_wait(N_CHUNKS - 2, 0)
    store_wait(N_CHUNKS - 1, 1)


def kernel(x, emb_table):
    packed = _pack_table(emb_table)
    pe = _make_pos_encoding()
    return _embed_kernel(x, packed, pe)
